# baseline (speedup 1.0000x reference)
import functools

import jax
import jax.numpy as jnp
from jax import lax
from jax.experimental import pallas as pl
from jax.experimental.pallas import tpu as pltpu
from jax.experimental.pallas import tpu_sc as plsc

B = 16384
A = 255
R = 8
NC, NS, L = 2, 16, 16
NW = NC * NS
ROWS_PER_W = B // NW
C = 64
NCHUNK = ROWS_PER_W // C
PENALTY = -1000000.0
COL_OFFS = tuple(L * k for k in range(A // L)) + (A - L,)


def _sc_body(q_hbm, past_hbm, ig_hbm, reach_hbm, out1_hbm, out2_hbm, pad_hbm,
             idx_v, feas_v, q_v, ig_v, out1_v, out2_v,
             blk_v, pblk_v, sem0, sem1):
    cid = lax.axis_index("c")
    sid = lax.axis_index("s")
    wid = sid * NC + cid
    iota = lax.iota(jnp.int32, L)
    sems = (sem0, sem1)

    start_row = jnp.minimum(sid * L, A - L)
    pltpu.sync_copy(reach_hbm.at[pl.ds(start_row, L), :], blk_v)
    for r in range(L):
        for col in COL_OFFS:
            pblk_v[r, pl.ds(col, L)] = blk_v[r, pl.ds(col, L)]
    pltpu.sync_copy(pblk_v, pad_hbm.at[pl.ds(start_row, L), :])
    plsc.subcore_barrier()

    def start(i, slot):
        base = wid * ROWS_PER_W + i * C
        pltpu.sync_copy(past_hbm.at[pl.ds(base, C)], idx_v.at[slot])
        g = pltpu.async_copy(pad_hbm.at[idx_v.at[slot]],
                             feas_v.at[slot], sems[slot])
        cq = pltpu.async_copy(q_hbm.at[pl.ds(base, C), :], q_v.at[slot],
                              sems[slot])
        ci = pltpu.async_copy(ig_hbm.at[pl.ds(base, C), :], ig_v.at[slot],
                              sems[slot])
        return g, cq, ci

    pend = start(0, 0)
    for i in range(NCHUNK):
        slot = i % 2
        base = wid * ROWS_PER_W + i * C
        cur = pend
        if i + 1 < NCHUNK:
            pend = start(i + 1, (i + 1) % 2)
        for h in cur:
            h.wait()
        qs, fs, igs = q_v.at[slot], feas_v.at[slot], ig_v.at[slot]

        def pre_body(r, carry):
            for col in COL_OFFS:
                qv = qs[r, pl.ds(col, L)]
                fv = fs[r, pl.ds(col, L)]
                qs[r, pl.ds(col, L)] = jnp.where(fv != 0, qv, qv + PENALTY)
            return carry

        lax.fori_loop(0, C, pre_body, 0, unroll=2)

        NG = C // L
        rows_g = [iota + g * L for g in range(NG)]

        def col_body(c, carry):
            ms, mis = carry
            csplat = jnp.full((L,), c, jnp.int32)
            new_ms, new_mis = [], []
            for g in range(NG):
                qp = plsc.load_gather(qs, [rows_g[g], csplat])
                better = qp > ms[g]
                new_ms.append(jnp.maximum(ms[g], qp))
                new_mis.append(jnp.where(better, csplat, mis[g]))
            return tuple(new_ms), tuple(new_mis)

        m0 = tuple(jnp.full((L,), -3.0e38, jnp.float32) for _ in range(NG))
        i0 = tuple(jnp.zeros((L,), jnp.int32) for _ in range(NG))
        _, avs = lax.fori_loop(0, A, col_body, (m0, i0), unroll=3)

        for g in range(C // L):
            rows = rows_g[g]
            av = avs[g]
            igp = jnp.zeros((L,), jnp.int32)
            for r in range(R):
                g_ig = plsc.load_gather(igs, [rows, jnp.full((L,), r, jnp.int32)])
                igp = igp | (g_ig << r)
            val = (av + 1) | igp
            out1_v[pl.ds(g * L, L)] = val - 1
            for r in range(R):
                bit = (val >> r) & 1
                plsc.store_scatter(out2_v, [rows, jnp.full((L,), r, jnp.int32)],
                                   bit)

        pltpu.sync_copy(out1_v, out1_hbm.at[pl.ds(base, C)])
        pltpu.sync_copy(out2_v, out2_hbm.at[pl.ds(base, C), :])


@jax.jit
def _routing_sc(q, past, ig, reach):
    mesh = plsc.VectorSubcoreMesh(core_axis_name="c", subcore_axis_name="s")
    f = functools.partial(
        pl.kernel,
        out_type=(
            jax.ShapeDtypeStruct((B,), jnp.int32),
            jax.ShapeDtypeStruct((B, R), jnp.int32),
            jax.ShapeDtypeStruct((256, 256), jnp.int32),
        ),
        mesh=mesh,
        compiler_params=pltpu.CompilerParams(
            needs_layout_passes=False, use_tc_tiling_on_sc=False),
        scratch_types=[
            pltpu.VMEM((2, C), jnp.int32),
            pltpu.VMEM((2, C, 256), jnp.int32),
            pltpu.VMEM((2, C, A), jnp.float32),
            pltpu.VMEM((2, C, R), jnp.int32),
            pltpu.VMEM((C,), jnp.int32),
            pltpu.VMEM((C, R), jnp.int32),
            pltpu.VMEM((L, A), jnp.int32),
            pltpu.VMEM((L, 256), jnp.int32),
            pltpu.SemaphoreType.DMA,
            pltpu.SemaphoreType.DMA,
        ],
    )(_sc_body)
    out1, out2, _ = f(q, past, ig, reach)
    return out1, out2


def kernel(q_table_predicted, input_ig_routing_matrix, is_warm_up_period,
           past_actions, action_space, reachability, action_space_reverse):
    del action_space, action_space_reverse
    out1, out2 = _routing_sc(q_table_predicted, past_actions,
                             input_ig_routing_matrix, reachability)
    warm = jnp.asarray(is_warm_up_period, jnp.int32) > 0
    out2 = jnp.where(warm, jnp.ones_like(out2), out2)
    return out1, out2


# speedup vs baseline: 1.6211x; 1.6211x over previous
import functools

import jax
import jax.numpy as jnp
from jax import lax
from jax.experimental import pallas as pl
from jax.experimental.pallas import tpu as pltpu
from jax.experimental.pallas import tpu_sc as plsc

B = 16384
A = 255
R = 8
NC, NS, L = 2, 16, 16
NW = NC * NS
ROWS_PER_W = B // NW
C = 64
NCHUNK = ROWS_PER_W // C
PENALTY = -1000000.0
COL_OFFS = tuple(L * k for k in range(A // L)) + (A - L,)


def _sc_body(q_hbm, past_hbm, ig_hbm, reach_hbm, out1_hbm, out2_hbm, pad_hbm,
             idx_v, feas_v, q_v, ig_v, a_v, out1_v, out2_v,
             blk_v, pblk_v, msc_m0, msc_i0, msc_m1, msc_i1, sem0, sem1):
    cid = lax.axis_index("c")
    sid = lax.axis_index("s")
    wid = sid * NC + cid
    iota = lax.iota(jnp.int32, L)
    sems = (sem0, sem1)

    start_row = jnp.minimum(sid * L, A - L)
    pltpu.sync_copy(reach_hbm.at[pl.ds(start_row, L), :], blk_v)
    for r in range(L):
        for col in COL_OFFS:
            pblk_v[r, pl.ds(col, L)] = blk_v[r, pl.ds(col, L)]
    pltpu.sync_copy(pblk_v, pad_hbm.at[pl.ds(start_row, L), :])
    plsc.subcore_barrier()

    def start(i, slot):
        base = wid * ROWS_PER_W + i * C
        pltpu.sync_copy(past_hbm.at[pl.ds(base, C)], idx_v.at[slot])
        g = pltpu.async_copy(pad_hbm.at[idx_v.at[slot]],
                             feas_v.at[slot], sems[slot])
        cq = pltpu.async_copy(q_hbm.at[pl.ds(base, C), :], q_v.at[slot],
                              sems[slot])
        ci = pltpu.async_copy(ig_hbm.at[pl.ds(base, C), :], ig_v.at[slot],
                              sems[slot])
        return g, cq, ci

    pend = start(0, 0)
    for i in range(NCHUNK):
        slot = i % 2
        base = wid * ROWS_PER_W + i * C
        cur = pend
        if i + 1 < NCHUNK:
            pend = start(i + 1, (i + 1) % 2)
        for h in cur:
            h.wait()
        qs, fs, igs = q_v.at[slot], feas_v.at[slot], ig_v.at[slot]

        def row_body(j, carry):
            res = []
            for p in range(2):
                r = 2 * j + p
                m = jnp.full((L,), -3.0e38, jnp.float32)
                mi = jnp.zeros((L,), jnp.int32)
                for col in COL_OFFS:
                    qv = qs[r, pl.ds(col, L)]
                    fv = fs[r, pl.ds(col, L)]
                    qp = jnp.where(fv != 0, qv, qv + PENALTY)
                    better = qp > m
                    m = jnp.maximum(m, qp)
                    mi = jnp.where(better, iota + col, mi)
                res.append((r, m, mi))
            scr = ((msc_m0, msc_i0), (msc_m1, msc_i1))
            for s in (8, 4, 2, 1):
                perm = iota ^ s
                nxt = []
                for (r, m, mi), (sm, si) in zip(res, scr):
                    sm[...] = m
                    si[...] = mi
                    ms = plsc.load_gather(sm, [perm])
                    is_ = plsc.load_gather(si, [perm])
                    take = (ms > m) | ((ms == m) & (is_ < mi))
                    nxt.append((r, jnp.maximum(m, ms),
                                jnp.where(take, is_, mi)))
                res = nxt
            for (r, m, mi) in res:
                plsc.store_scatter(a_v, [jnp.full((L,), r, jnp.int32)], mi,
                                   mask=iota == 0)
            return carry

        lax.fori_loop(0, C // 2, row_body, 0)

        for g in range(C // L):
            rows = iota + g * L
            av = a_v[pl.ds(g * L, L)]
            igp = jnp.zeros((L,), jnp.int32)
            for r in range(R):
                g_ig = plsc.load_gather(igs, [rows, jnp.full((L,), r, jnp.int32)])
                igp = igp | (g_ig << r)
            val = (av + 1) | igp
            out1_v[pl.ds(g * L, L)] = val - 1
            for r in range(R):
                bit = (val >> r) & 1
                plsc.store_scatter(out2_v, [rows, jnp.full((L,), r, jnp.int32)],
                                   bit)

        pltpu.sync_copy(out1_v, out1_hbm.at[pl.ds(base, C)])
        pltpu.sync_copy(out2_v, out2_hbm.at[pl.ds(base, C), :])


@jax.jit
def _routing_sc(q, past, ig, reach):
    mesh = plsc.VectorSubcoreMesh(core_axis_name="c", subcore_axis_name="s")
    f = functools.partial(
        pl.kernel,
        out_type=(
            jax.ShapeDtypeStruct((B,), jnp.int32),
            jax.ShapeDtypeStruct((B, R), jnp.int32),
            jax.ShapeDtypeStruct((256, 256), jnp.int32),
        ),
        mesh=mesh,
        compiler_params=pltpu.CompilerParams(
            needs_layout_passes=False, use_tc_tiling_on_sc=False),
        scratch_types=[
            pltpu.VMEM((2, C), jnp.int32),
            pltpu.VMEM((2, C, 256), jnp.int32),
            pltpu.VMEM((2, C, A), jnp.float32),
            pltpu.VMEM((2, C, R), jnp.int32),
            pltpu.VMEM((C,), jnp.int32),
            pltpu.VMEM((C,), jnp.int32),
            pltpu.VMEM((C, R), jnp.int32),
            pltpu.VMEM((L, A), jnp.int32),
            pltpu.VMEM((L, 256), jnp.int32),
            pltpu.VMEM((L,), jnp.float32),
            pltpu.VMEM((L,), jnp.int32),
            pltpu.VMEM((L,), jnp.float32),
            pltpu.VMEM((L,), jnp.int32),
            pltpu.SemaphoreType.DMA,
            pltpu.SemaphoreType.DMA,
        ],
    )(_sc_body)
    out1, out2, _ = f(q, past, ig, reach)
    return out1, out2


def kernel(q_table_predicted, input_ig_routing_matrix, is_warm_up_period,
           past_actions, action_space, reachability, action_space_reverse):
    del action_space, action_space_reverse
    out1, out2 = _routing_sc(q_table_predicted, past_actions,
                             input_ig_routing_matrix, reachability)
    warm = jnp.asarray(is_warm_up_period, jnp.int32) > 0
    out2 = jnp.where(warm, jnp.ones_like(out2), out2)
    return out1, out2


# speedup vs baseline: 1.6465x; 1.0156x over previous
import functools

import jax
import jax.numpy as jnp
from jax import lax
from jax.experimental import pallas as pl
from jax.experimental.pallas import tpu as pltpu
from jax.experimental.pallas import tpu_sc as plsc

B = 16384
A = 255
R = 8
NC, NS, L = 2, 16, 16
NW = NC * NS
ROWS_PER_W = B // NW
C = 64
NCHUNK = ROWS_PER_W // C
PENALTY = -1000000.0
COL_OFFS = tuple(L * k for k in range(A // L)) + (A - L,)


def _sc_body(q_hbm, past_hbm, ig_hbm, reach_hbm, out1_hbm, out2_hbm, pad_hbm,
             idx_v, feas_v, q_v, ig_v, a_v, out1_v, out2_v,
             blk_v, pblk_v, msc_m0, msc_i0, msc_m1, msc_i1,
             msc_m2, msc_i2, msc_m3, msc_i3, sem0, sem1):
    cid = lax.axis_index("c")
    sid = lax.axis_index("s")
    wid = sid * NC + cid
    iota = lax.iota(jnp.int32, L)
    sems = (sem0, sem1)

    start_row = jnp.minimum(sid * L, A - L)
    pltpu.sync_copy(reach_hbm.at[pl.ds(start_row, L), :], blk_v)
    for r in range(L):
        for col in COL_OFFS:
            pblk_v[r, pl.ds(col, L)] = blk_v[r, pl.ds(col, L)]
    pltpu.sync_copy(pblk_v, pad_hbm.at[pl.ds(start_row, L), :])
    plsc.subcore_barrier()

    def start(i, slot):
        base = wid * ROWS_PER_W + i * C
        pltpu.sync_copy(past_hbm.at[pl.ds(base, C)], idx_v.at[slot])
        g = pltpu.async_copy(pad_hbm.at[idx_v.at[slot]],
                             feas_v.at[slot], sems[slot])
        cq = pltpu.async_copy(q_hbm.at[pl.ds(base, C), :], q_v.at[slot],
                              sems[slot])
        ci = pltpu.async_copy(ig_hbm.at[pl.ds(base, C), :], ig_v.at[slot],
                              sems[slot])
        return g, cq, ci

    pend = start(0, 0)
    for i in range(NCHUNK):
        slot = i % 2
        base = wid * ROWS_PER_W + i * C
        cur = pend
        if i + 1 < NCHUNK:
            pend = start(i + 1, (i + 1) % 2)
        for h in cur:
            h.wait()
        qs, fs, igs = q_v.at[slot], feas_v.at[slot], ig_v.at[slot]

        def row_body(j, carry):
            res = []
            for p in range(4):
                r = 4 * j + p
                m = jnp.full((L,), -3.0e38, jnp.float32)
                mi = jnp.zeros((L,), jnp.int32)
                for col in COL_OFFS:
                    qv = qs[r, pl.ds(col, L)]
                    fv = fs[r, pl.ds(col, L)]
                    qp = jnp.where(fv != 0, qv, qv + PENALTY)
                    better = qp > m
                    m = jnp.maximum(m, qp)
                    mi = jnp.where(better, iota + col, mi)
                res.append((r, m, mi))
            scr = ((msc_m0, msc_i0), (msc_m1, msc_i1),
                   (msc_m2, msc_i2), (msc_m3, msc_i3))
            for s in (8, 4, 2, 1):
                perm = iota ^ s
                nxt = []
                for (r, m, mi), (sm, si) in zip(res, scr):
                    sm[...] = m
                    si[...] = mi
                    ms = plsc.load_gather(sm, [perm])
                    is_ = plsc.load_gather(si, [perm])
                    take = (ms > m) | ((ms == m) & (is_ < mi))
                    nxt.append((r, jnp.maximum(m, ms),
                                jnp.where(take, is_, mi)))
                res = nxt
            for (r, m, mi) in res:
                plsc.store_scatter(a_v, [jnp.full((L,), r, jnp.int32)], mi,
                                   mask=iota == 0)
            return carry

        lax.fori_loop(0, C // 4, row_body, 0)

        for g in range(C // L):
            rows = iota + g * L
            av = a_v[pl.ds(g * L, L)]
            igp = jnp.zeros((L,), jnp.int32)
            for r in range(R):
                g_ig = plsc.load_gather(igs, [rows, jnp.full((L,), r, jnp.int32)])
                igp = igp | (g_ig << r)
            val = (av + 1) | igp
            out1_v[pl.ds(g * L, L)] = val - 1
            for r in range(R):
                bit = (val >> r) & 1
                plsc.store_scatter(out2_v, [rows, jnp.full((L,), r, jnp.int32)],
                                   bit)

        pltpu.sync_copy(out1_v, out1_hbm.at[pl.ds(base, C)])
        pltpu.sync_copy(out2_v, out2_hbm.at[pl.ds(base, C), :])


@jax.jit
def _routing_sc(q, past, ig, reach):
    mesh = plsc.VectorSubcoreMesh(core_axis_name="c", subcore_axis_name="s")
    f = functools.partial(
        pl.kernel,
        out_type=(
            jax.ShapeDtypeStruct((B,), jnp.int32),
            jax.ShapeDtypeStruct((B, R), jnp.int32),
            jax.ShapeDtypeStruct((256, 256), jnp.int32),
        ),
        mesh=mesh,
        compiler_params=pltpu.CompilerParams(
            needs_layout_passes=False, use_tc_tiling_on_sc=False),
        scratch_types=[
            pltpu.VMEM((2, C), jnp.int32),
            pltpu.VMEM((2, C, 256), jnp.int32),
            pltpu.VMEM((2, C, A), jnp.float32),
            pltpu.VMEM((2, C, R), jnp.int32),
            pltpu.VMEM((C,), jnp.int32),
            pltpu.VMEM((C,), jnp.int32),
            pltpu.VMEM((C, R), jnp.int32),
            pltpu.VMEM((L, A), jnp.int32),
            pltpu.VMEM((L, 256), jnp.int32),
            pltpu.VMEM((L,), jnp.float32),
            pltpu.VMEM((L,), jnp.int32),
            pltpu.VMEM((L,), jnp.float32),
            pltpu.VMEM((L,), jnp.int32),
            pltpu.VMEM((L,), jnp.float32),
            pltpu.VMEM((L,), jnp.int32),
            pltpu.VMEM((L,), jnp.float32),
            pltpu.VMEM((L,), jnp.int32),
            pltpu.SemaphoreType.DMA,
            pltpu.SemaphoreType.DMA,
        ],
    )(_sc_body)
    out1, out2, _ = f(q, past, ig, reach)
    return out1, out2


def kernel(q_table_predicted, input_ig_routing_matrix, is_warm_up_period,
           past_actions, action_space, reachability, action_space_reverse):
    del action_space, action_space_reverse
    out1, out2 = _routing_sc(q_table_predicted, past_actions,
                             input_ig_routing_matrix, reachability)
    warm = jnp.asarray(is_warm_up_period, jnp.int32) > 0
    out2 = jnp.where(warm, jnp.ones_like(out2), out2)
    return out1, out2
